# Initial kernel scaffold; baseline (speedup 1.0000x reference)
import jax
import jax.numpy as jnp
from jax.experimental import pallas as pl


def kernel(x, edge_index, batch, W1, b1, g1, be1, W2, b2, g2, be2, W3, b3, g3, be3, att_w, Wc, bc):
    raise NotImplementedError("write your pallas kernel here")


# speedup vs baseline: 10.5063x; 10.5063x over previous
import functools

import jax
import jax.numpy as jnp
import numpy as np
from jax import lax
from jax.experimental import pallas as pl
from jax.experimental.pallas import tpu as pltpu
from jax.experimental.pallas import tpu_sc as plsc

N = 10000
E = 320000
D = 128
G = 64
BN_EPS = 1e-5

NC = 2
NS = 16
NW = NC * NS
CHUNK = 128
CH = 79
EPT = CH * CHUNK
EPAD = NW * EPT
RP = 10112
RPT = RP // NS

BLK = 2000
NBLK = N // BLK

@functools.lru_cache(maxsize=None)
def _mesh():
    return plsc.VectorSubcoreMesh(core_axis_name="c", subcore_axis_name="s",
                                  num_cores=NC, num_subcores=NS)


def _deg_body(dstr, ones_hbm, z_hbm, out, idx_d, ones_v, acc, sem):
    c = lax.axis_index("c")
    s = lax.axis_index("s")
    w = c * NS + s
    base = s * RPT
    pltpu.sync_copy(dstr.at[w], idx_d)
    pltpu.sync_copy(ones_hbm, ones_v)
    pltpu.sync_copy(z_hbm, acc.at[pl.ds(base, RPT)])
    plsc.subcore_barrier()

    def body(j, carry):
        pltpu.sync_copy(ones_v, acc.at[idx_d.at[j, 0]], add=True)
        return carry

    lax.fori_loop(0, CH, body, 0)
    plsc.subcore_barrier()
    pltpu.sync_copy(acc.at[pl.ds(base, RPT)], out.at[c, pl.ds(base, RPT)])


@functools.lru_cache(maxsize=None)
def _deg_kernel():
    return pl.kernel(
        _deg_body,
        out_type=jax.ShapeDtypeStruct((NC, RP, 16), jnp.float32),
        mesh=_mesh(),
        scratch_types=[
            pltpu.VMEM((CH, 1, CHUNK), jnp.int32),
            pltpu.VMEM((CHUNK, 16), jnp.float32),
            pltpu.VMEM_SHARED((RP, 16), jnp.float32),
            pltpu.SemaphoreType.DMA,
        ],
    )


def _deg_call(dstr, ones16, z16):
    return _deg_kernel()(dstr, ones16, z16)


def _spmm_body(table, srcr, dstr, z_hbm, out, idx_s, idx_d, rows, acc, sem):
    c = lax.axis_index("c")
    s = lax.axis_index("s")
    w = c * NS + s
    base = s * RPT
    pltpu.sync_copy(srcr.at[w], idx_s)
    pltpu.sync_copy(dstr.at[w], idx_d)

    @pl.when(c == 0)
    def _():
        pltpu.sync_copy(table.at[pl.ds(base, RPT)], acc.at[pl.ds(base, RPT)])

    @pl.when(c == 1)
    def _():
        pltpu.sync_copy(z_hbm, acc.at[pl.ds(base, RPT)])

    plsc.subcore_barrier()

    def body(j, carry):
        pltpu.async_copy(table.at[idx_s.at[j, 0]], rows, sem).wait()
        pltpu.sync_copy(rows, acc.at[idx_d.at[j, 0]], add=True)
        return carry

    lax.fori_loop(0, CH, body, 0)
    plsc.subcore_barrier()
    pltpu.sync_copy(acc.at[pl.ds(base, RPT)], out.at[c, pl.ds(base, RPT)])


@functools.lru_cache(maxsize=None)
def _spmm_kernel():
    return pl.kernel(
        _spmm_body,
        out_type=jax.ShapeDtypeStruct((NC, RP, D), jnp.float32),
        mesh=_mesh(),
        scratch_types=[
            pltpu.VMEM((CH, 1, CHUNK), jnp.int32),
            pltpu.VMEM((CH, 1, CHUNK), jnp.int32),
            pltpu.VMEM((CHUNK, D), jnp.float32),
            pltpu.VMEM_SHARED((RP, D), jnp.float32),
            pltpu.SemaphoreType.DMA,
        ],
    )


def _spmm_call(table, srcr, dstr, zd):
    return _spmm_kernel()(table, srcr, dstr, zd)


_BN_SCALE = float(1.0 / np.sqrt(np.float32(1.0 + BN_EPS), dtype=np.float32))


def _tc_prep_body(x_ref, w_ref, deg_ref, hw_ref, dinv_ref):
    deg = deg_ref[0, :, 0:1] + deg_ref[1, :, 0:1] + 1.0
    dinv = lax.rsqrt(deg)
    xw = jnp.dot(x_ref[...], w_ref[...], preferred_element_type=jnp.float32)
    hw_ref[...] = xw * dinv
    dinv_ref[...] = jnp.broadcast_to(dinv, (BLK, D))


def _tc_prep(x, w1, deg):
    return pl.pallas_call(
        _tc_prep_body,
        grid=(NBLK,),
        in_specs=[
            pl.BlockSpec((BLK, D), lambda i: (i, 0)),
            pl.BlockSpec((D, D), lambda i: (0, 0)),
            pl.BlockSpec((NC, BLK, 16), lambda i: (0, i, 0)),
        ],
        out_specs=[
            pl.BlockSpec((BLK, D), lambda i: (i, 0)),
            pl.BlockSpec((BLK, D), lambda i: (i, 0)),
        ],
        out_shape=[
            jax.ShapeDtypeStruct((RP, D), jnp.float32),
            jax.ShapeDtypeStruct((N, D), jnp.float32),
        ],
    )(x, w1, deg)


def _tc_mid_body(agg_ref, dinv_ref, b_ref, g_ref, be_ref, w_ref, hw_ref):
    dinv = dinv_ref[...]
    conv = dinv * (agg_ref[0] + agg_ref[1]) + b_ref[...]
    y = jax.nn.relu(g_ref[...] * (conv * _BN_SCALE) + be_ref[...])
    hw_ref[...] = jnp.dot(y, w_ref[...], preferred_element_type=jnp.float32) * dinv


def _tc_mid(agg, dinvb, b, g, be, w_next):
    return pl.pallas_call(
        _tc_mid_body,
        grid=(NBLK,),
        in_specs=[
            pl.BlockSpec((NC, BLK, D), lambda i: (0, i, 0)),
            pl.BlockSpec((BLK, D), lambda i: (i, 0)),
            pl.BlockSpec((1, D), lambda i: (0, 0)),
            pl.BlockSpec((1, D), lambda i: (0, 0)),
            pl.BlockSpec((1, D), lambda i: (0, 0)),
            pl.BlockSpec((D, D), lambda i: (0, 0)),
        ],
        out_specs=pl.BlockSpec((BLK, D), lambda i: (i, 0)),
        out_shape=jax.ShapeDtypeStruct((RP, D), jnp.float32),
    )(agg, dinvb, b, g, be, w_next)


def _tc_fin_body(agg_ref, dinv_ref, b_ref, g_ref, be_ref, attw_ref, wc_ref,
                 bc_ref, bat_ref, logits_ref, att_ref, acc_ref):
    i = pl.program_id(0)
    dinv = dinv_ref[...]
    conv = dinv * (agg_ref[0] + agg_ref[1]) + b_ref[...]
    y = jax.nn.relu(g_ref[...] * (conv * _BN_SCALE) + be_ref[...])
    att = jax.nn.sigmoid(jnp.dot(y, attw_ref[...],
                                 preferred_element_type=jnp.float32))
    att_ref[...] = att
    wtd = y * att
    bat = bat_ref[0, 0, :]
    oh = (bat[:, None] == lax.broadcasted_iota(jnp.int32, (BLK, G), 1)
          ).astype(jnp.float32)
    part = lax.dot_general(oh, wtd, (((0,), (0,)), ((), ())),
                           preferred_element_type=jnp.float32)

    @pl.when(i == 0)
    def _():
        acc_ref[...] = part

    @pl.when(i > 0)
    def _():
        acc_ref[...] += part

    @pl.when(i == NBLK - 1)
    def _():
        logits_ref[...] = (jnp.dot(acc_ref[...], wc_ref[...],
                                   preferred_element_type=jnp.float32)
                           + bc_ref[...])


def _tc_fin(agg, dinvb, b, g, be, att_w, wc, bc, bat3):
    return pl.pallas_call(
        _tc_fin_body,
        grid=(NBLK,),
        in_specs=[
            pl.BlockSpec((NC, BLK, D), lambda i: (0, i, 0)),
            pl.BlockSpec((BLK, D), lambda i: (i, 0)),
            pl.BlockSpec((1, D), lambda i: (0, 0)),
            pl.BlockSpec((1, D), lambda i: (0, 0)),
            pl.BlockSpec((1, D), lambda i: (0, 0)),
            pl.BlockSpec((D, 1), lambda i: (0, 0)),
            pl.BlockSpec((D, 1), lambda i: (0, 0)),
            pl.BlockSpec((1, 1), lambda i: (0, 0)),
            pl.BlockSpec((1, 1, BLK), lambda i: (i, 0, 0)),
        ],
        out_specs=[
            pl.BlockSpec((G, 1), lambda i: (0, 0)),
            pl.BlockSpec((BLK, 1), lambda i: (i, 0)),
        ],
        out_shape=[
            jax.ShapeDtypeStruct((G, 1), jnp.float32),
            jax.ShapeDtypeStruct((N, 1), jnp.float32),
        ],
        scratch_shapes=[pltpu.VMEM((G, D), jnp.float32)],
    )(agg, dinvb, b, g, be, att_w, wc, bc, bat3)


def kernel(x, edge_index, batch, W1, b1, g1, be1, W2, b2, g2, be2,
           W3, b3, g3, be3, att_w, Wc, bc):
    src = edge_index[0]
    dst = edge_index[1]
    pad = EPAD - E
    srcr = jnp.concatenate(
        [src, jnp.zeros((pad,), jnp.int32)]).reshape(NW, CH, 1, CHUNK)
    dstr = jnp.concatenate(
        [dst, jnp.full((pad,), N, jnp.int32)]).reshape(NW, CH, 1, CHUNK)

    ones16 = jnp.ones((CHUNK, 16), jnp.float32)
    z16 = jnp.zeros((RPT, 16), jnp.float32)
    zd = jnp.zeros((RPT, D), jnp.float32)

    deg = _deg_call(dstr, ones16, z16)

    hw1, dinvb = _tc_prep(x, W1, deg)
    agg1 = _spmm_call(hw1, srcr, dstr, zd)
    hw2 = _tc_mid(agg1, dinvb, b1.reshape(1, D), g1.reshape(1, D),
                  be1.reshape(1, D), W2)
    agg2 = _spmm_call(hw2, srcr, dstr, zd)
    hw3 = _tc_mid(agg2, dinvb, b2.reshape(1, D), g2.reshape(1, D),
                  be2.reshape(1, D), W3)
    agg3 = _spmm_call(hw3, srcr, dstr, zd)

    bat3 = batch.reshape(NBLK, 1, BLK)
    logits, att = _tc_fin(agg3, dinvb, b3.reshape(1, D), g3.reshape(1, D),
                          be3.reshape(1, D), att_w, Wc, bc.reshape(1, 1), bat3)
    return (logits, att)
